# baseline (speedup 1.0000x reference)
import functools

import jax
import jax.numpy as jnp
from jax import lax
from jax.experimental import pallas as pl
from jax.experimental.pallas import tpu as pltpu
from jax.experimental.pallas import tpu_sc as plsc

NSC = 2
NTILES = 16
LANES = 16


def _dual_linear(h, bias_prev, wlT, bl, wrT, br, do_relu, block_m=1000):
    n, d = h.shape

    def body(h_ref, bp_ref, wl_ref, bl_ref, wr_ref, br_ref, xl_ref, xr_ref):
        hh = h_ref[...]
        if do_relu:
            hh = jnp.maximum(hh + bp_ref[...], 0.0)
        xl_ref[...] = (
            jnp.dot(hh, wl_ref[...], preferred_element_type=jnp.float32)
            + bl_ref[...]
        )
        xr_ref[...] = (
            jnp.dot(hh, wr_ref[...], preferred_element_type=jnp.float32)
            + br_ref[...]
        )

    grid = (n // block_m,)
    return pl.pallas_call(
        body,
        grid=grid,
        in_specs=[
            pl.BlockSpec((block_m, d), lambda i: (i, 0)),
            pl.BlockSpec((1, d), lambda i: (0, 0)),
            pl.BlockSpec((d, d), lambda i: (0, 0)),
            pl.BlockSpec((1, d), lambda i: (0, 0)),
            pl.BlockSpec((d, d), lambda i: (0, 0)),
            pl.BlockSpec((1, d), lambda i: (0, 0)),
        ],
        out_specs=[
            pl.BlockSpec((block_m, d), lambda i: (i, 0)),
            pl.BlockSpec((block_m, d), lambda i: (i, 0)),
        ],
        out_shape=[
            jax.ShapeDtypeStruct((n, d), jnp.float32),
            jax.ShapeDtypeStruct((n, d), jnp.float32),
        ],
    )(h, bias_prev, wlT, bl, wrT, br)


def _bias_add(h, b, block_m=1000):
    n, d = h.shape

    def body(h_ref, b_ref, o_ref):
        o_ref[...] = h_ref[...] + b_ref[...]

    return pl.pallas_call(
        body,
        grid=(n // block_m,),
        in_specs=[
            pl.BlockSpec((block_m, d), lambda i: (i, 0)),
            pl.BlockSpec((1, d), lambda i: (0, 0)),
        ],
        out_specs=pl.BlockSpec((block_m, d), lambda i: (i, 0)),
        out_shape=jax.ShapeDtypeStruct((n, d), jnp.float32),
    )(h, b)


def _gat_edge_sc(xl, xr, src, dst, ew, we, att):
    n, d = xl.shape
    e = src.shape[0]
    assert d % LANES == 0 and n % NSC == 0 and e % NTILES == 0
    half = n // NSC
    halfp = ((half + 319) // 320) * 320
    chunk = e // NTILES
    assert chunk % 16 == 0
    B = 16
    cap = chunk + B + 16
    nd = d // LANES
    rows_pt = halfp // NTILES
    assert rows_pt % 8 == 0 and half % 8 == 0

    mesh = plsc.VectorSubcoreMesh(core_axis_name="c", subcore_axis_name="s")

    @functools.partial(
        pl.kernel,
        out_type=jax.ShapeDtypeStruct((n, d), jnp.float32),
        mesh=mesh,
        compiler_params=pltpu.CompilerParams(needs_layout_passes=False),
        scratch_types=[
            pltpu.VMEM((cap,), jnp.int32),
            pltpu.VMEM((cap,), jnp.int32),
            pltpu.VMEM((2 * B, d), jnp.float32),
            pltpu.VMEM((B, d), jnp.float32),
            pltpu.VMEM((4 * B, 128), jnp.int32),
            pltpu.VMEM((B,), jnp.int32),
            pltpu.VMEM((B + 16,), jnp.float32),
            pltpu.VMEM((B + 16,), jnp.float32),
            pltpu.VMEM((B + 16,), jnp.int32),
            pltpu.VMEM((B,), jnp.int32),
            pltpu.VMEM((d,), jnp.float32),
            pltpu.VMEM((d,), jnp.float32),
            pltpu.VMEM((48,), jnp.float32),
            pltpu.VMEM((2048,), jnp.float32),
            pltpu.VMEM_SHARED((halfp,), jnp.float32),
            pltpu.VMEM_SHARED((halfp * d,), jnp.float32),
            pltpu.SemaphoreType.DMA,
            pltpu.SemaphoreType.DMA,
            pltpu.SemaphoreType.DMA,
            pltpu.SemaphoreType.DMA,
        ],
    )
    def k(xl_hbm, xr_hbm, src_hbm, dst_hbm, ew_hbm, we_hbm, att_hbm, out_hbm,
          eidx_c, dst_c, bufL, bufR, elem_idx, src_b, w_b, a_b, didx, didx_s,
          att_v, we_v, dnbuf, zbuf, denom_sp, out_sp, sem1, sem2, sem3, sem4):
        sc = lax.axis_index("c")
        tl = lax.axis_index("s")
        half_base = sc * half
        iota16 = lax.iota(jnp.int32, LANES)
        zf16 = jnp.zeros((LANES,), jnp.float32)

        for c in range(2048 // LANES):
            zbuf[pl.ds(c * LANES, LANES)] = zf16

        @pl.loop(0, rows_pt * d // 2048)
        def _(j):
            pltpu.sync_copy(
                zbuf, out_sp.at[pl.ds((tl * rows_pt) * d + j * 2048, 2048)])

        pltpu.sync_copy(zbuf.at[pl.ds(0, rows_pt)],
                        denom_sp.at[pl.ds(tl * rows_pt, rows_pt)])

        pltpu.sync_copy(att_hbm, att_v)
        pltpu.sync_copy(we_hbm, we_v)
        att_regs = [att_v[pl.ds(c * LANES, LANES)] for c in range(nd)]
        we_regs = [we_v[pl.ds(c * LANES, LANES)] for c in range(nd)]

        ebase = tl * chunk
        pltpu.sync_copy(dst_hbm.at[pl.ds(ebase, chunk)],
                        dst_c.at[pl.ds(0, chunk)])

        plsc.subcore_barrier()

        @pl.loop(0, chunk // 16, init_carry=jnp.int32(0))
        def cnt(i, cnt):
            off = i * 16
            dd = dst_c[pl.ds(off, 16)]
            m = (dd >= half_base) & (dd < half_base + half)
            plsc.store_compressed(eidx_c.at[pl.ds(cnt, 16)],
                                  ebase + off + iota16, mask=m)
            plsc.store_compressed(dst_c.at[pl.ds(cnt, 16)], dd, mask=m)
            pc = plsc.all_reduce_population_count(m)
            return cnt + jnp.max(pc)

        for j in range(2):
            eidx_c[pl.ds(cnt + j * 16, 16)] = jnp.zeros((16,), jnp.int32)
            dst_c[pl.ds(cnt + j * 16, 16)] = (
                jnp.full((16,), 1, jnp.int32) * half_base)
        nb = (cnt + B - 1) // B

        def process_block(b, p):
            off = b * B
            rb = p * B

            @pl.when(b >= 2)
            def _():
                pltpu.make_async_copy(
                    xl_hbm.at[pl.ds(0, B)],
                    bufL.at[pl.ds(rb, B)], sem4).wait()

            cp1 = pltpu.async_copy(
                src_hbm.at[eidx_c.at[pl.ds(off, B)]], src_b, sem1)
            cp2 = pltpu.async_copy(
                ew_hbm.at[eidx_c.at[pl.ds(off, B)]], w_b.at[pl.ds(0, B)], sem2)
            cp3 = pltpu.async_copy(
                xr_hbm.at[dst_c.at[pl.ds(off, B)]], bufR, sem3)
            cp1.wait()
            cp1 = pltpu.async_copy(
                xl_hbm.at[src_b], bufL.at[pl.ds(rb, B)], sem1)
            cp2.wait()
            cp3.wait()
            cp1.wait()

            @pl.loop(0, 16, init_carry=zf16)
            def logits(ei, logits):
                w_s = w_b[pl.ds(ei, 16)][0]
                acc = zf16
                for c in range(nd):
                    sl = pl.ds(c * LANES, LANES)
                    v = bufL[rb + ei, sl] + bufR[ei, sl] + w_s * we_regs[c]
                    v = jnp.maximum(v, 0.2 * v)
                    acc = acc + att_regs[c] * v
                lg = jnp.sum(acc)
                return jnp.where(iota16 == ei, lg, logits)

            tmask = (off + iota16) < cnt
            a_b[pl.ds(0, 16)] = jnp.where(tmask, jnp.exp(logits), 0.0)
            dl = dst_c[pl.ds(off, 16)] - half_base
            dlm = jnp.where(tmask, dl, 0)
            didx[pl.ds(0, 16)] = dlm
            didx_s[pl.ds(0, 16)] = dlm

            @pl.loop(0, B)
            def _(ei):
                a_s = a_b[pl.ds(ei, 16)][0]
                for c in range(nd):
                    sl = pl.ds(c * LANES, LANES)
                    bufL[rb + ei, sl] = a_s * bufL[rb + ei, sl]
                base = didx[pl.ds(ei, 16)][0] * d
                for c in range(nd):
                    elem_idx[2 * (rb + ei) + (c // 8),
                             pl.ds((c % 8) * 16, 16)] = (
                        base + c * LANES + iota16)

            for ei in range(B):
                pltpu.async_copy(
                    bufL.at[rb + ei, pl.ds(0, 128)],
                    out_sp.at[elem_idx.at[2 * (rb + ei)]], sem4, add=True)
                pltpu.async_copy(
                    bufL.at[rb + ei, pl.ds(128, 128)],
                    out_sp.at[elem_idx.at[2 * (rb + ei) + 1]], sem4, add=True)
            pltpu.sync_copy(a_b.at[pl.ds(0, B)],
                            denom_sp.at[didx_s], add=True)

        @pl.loop(0, nb // 2)
        def _(j):
            process_block(2 * j, 0)
            process_block(2 * j + 1, 1)

        @pl.when(nb % 2 == 1)
        def _():
            process_block(nb - 1, 0)

        @pl.when(nb >= 1)
        def _():
            pltpu.make_async_copy(
                xl_hbm.at[pl.ds(0, B)], bufL.at[pl.ds(0, B)], sem4).wait()

        @pl.when(nb >= 2)
        def _():
            pltpu.make_async_copy(
                xl_hbm.at[pl.ds(0, B)], bufL.at[pl.ds(B, B)], sem4).wait()

        plsc.subcore_barrier()

        nchunks = half // 8
        @pl.loop(0, (nchunks + NTILES - 1) // NTILES)
        def _(j):
            idx = tl + j * NTILES

            @pl.when(idx < nchunks)
            def _():
                pltpu.sync_copy(denom_sp.at[pl.ds(idx * 8, 8)],
                                dnbuf.at[pl.ds(0, 8)])
                for r in range(8):
                    pltpu.sync_copy(
                        out_sp.at[pl.ds((idx * 8 + r) * d, d)],
                        bufR.at[r])
                dnbuf[pl.ds(16, 16)] = 1.0 / (dnbuf[pl.ds(0, 16)] + 1e-16)
                for r in range(8):
                    inv_s = dnbuf[pl.ds(16 + r, 16)][0]
                    for c in range(nd):
                        sl = pl.ds(c * LANES, LANES)
                        bufR[r, sl] = inv_s * bufR[r, sl]
                pltpu.sync_copy(
                    bufR.at[pl.ds(0, 8)],
                    out_hbm.at[pl.ds(half_base + idx * 8, 8)])

    return k(xl, xr, src, dst, ew, we, att)


def kernel(x, edge_index, edge_weight, params):
    n, d = x.shape
    src = edge_index[0]
    dst = edge_index[1]
    nl = len(params)
    h = x
    zeros_d = jnp.zeros((d,), jnp.float32)
    for i, p in enumerate(params):
        bias_prev = params[i - 1]["bias"] if i > 0 else zeros_d
        xl, xr = _dual_linear(
            h,
            bias_prev.reshape(1, d),
            p["Wl"].T,
            p["bl"].reshape(1, d),
            p["Wr"].T,
            p["br"].reshape(1, d),
            do_relu=(i > 0),
        )
        we_eff = p["We"][:, 0] if i < nl - 1 else zeros_d
        h = _gat_edge_sc(xl, xr, src, dst, edge_weight, we_eff, p["att"])
    return _bias_add(h, params[-1]["bias"].reshape(1, d))


# speedup vs baseline: 1.2608x; 1.2608x over previous
import functools

import jax
import jax.numpy as jnp
from jax import lax
from jax.experimental import pallas as pl
from jax.experimental.pallas import tpu as pltpu
from jax.experimental.pallas import tpu_sc as plsc

NSC = 2
NTILES = 16
LANES = 16


def _dual_linear(h, bias_prev, wlT, bl, wrT, br, do_relu, block_m=1000):
    n, d = h.shape

    def body(h_ref, bp_ref, wl_ref, bl_ref, wr_ref, br_ref, xl_ref, xr_ref):
        hh = h_ref[...]
        if do_relu:
            hh = jnp.maximum(hh + bp_ref[...], 0.0)
        xl_ref[...] = (
            jnp.dot(hh, wl_ref[...], preferred_element_type=jnp.float32)
            + bl_ref[...]
        )
        xr_ref[...] = (
            jnp.dot(hh, wr_ref[...], preferred_element_type=jnp.float32)
            + br_ref[...]
        )

    grid = (n // block_m,)
    return pl.pallas_call(
        body,
        grid=grid,
        in_specs=[
            pl.BlockSpec((block_m, d), lambda i: (i, 0)),
            pl.BlockSpec((1, d), lambda i: (0, 0)),
            pl.BlockSpec((d, d), lambda i: (0, 0)),
            pl.BlockSpec((1, d), lambda i: (0, 0)),
            pl.BlockSpec((d, d), lambda i: (0, 0)),
            pl.BlockSpec((1, d), lambda i: (0, 0)),
        ],
        out_specs=[
            pl.BlockSpec((block_m, d), lambda i: (i, 0)),
            pl.BlockSpec((block_m, d), lambda i: (i, 0)),
        ],
        out_shape=[
            jax.ShapeDtypeStruct((n, d), jnp.float32),
            jax.ShapeDtypeStruct((n, d), jnp.float32),
        ],
    )(h, bias_prev, wlT, bl, wrT, br)


def _bias_add(h, b, block_m=1000):
    n, d = h.shape

    def body(h_ref, b_ref, o_ref):
        o_ref[...] = h_ref[...] + b_ref[...]

    return pl.pallas_call(
        body,
        grid=(n // block_m,),
        in_specs=[
            pl.BlockSpec((block_m, d), lambda i: (i, 0)),
            pl.BlockSpec((1, d), lambda i: (0, 0)),
        ],
        out_specs=pl.BlockSpec((block_m, d), lambda i: (i, 0)),
        out_shape=jax.ShapeDtypeStruct((n, d), jnp.float32),
    )(h, b)


def _gat_edge_sc(xl, xr, src, dst, ew, we, att):
    n, d = xl.shape
    e = src.shape[0]
    assert d % LANES == 0 and n % NSC == 0 and e % NTILES == 0
    half = n // NSC
    halfp = ((half + 319) // 320) * 320
    chunk = e // NTILES
    assert chunk % 16 == 0
    B = 16
    cap = chunk + B + 16
    nd = d // LANES
    rows_pt = halfp // NTILES
    assert rows_pt % 8 == 0 and half % 8 == 0

    mesh = plsc.VectorSubcoreMesh(core_axis_name="c", subcore_axis_name="s")

    @functools.partial(
        pl.kernel,
        out_type=jax.ShapeDtypeStruct((n, d), jnp.float32),
        mesh=mesh,
        compiler_params=pltpu.CompilerParams(needs_layout_passes=False),
        scratch_types=[
            pltpu.VMEM((cap,), jnp.int32),
            pltpu.VMEM((cap,), jnp.int32),
            pltpu.VMEM((2 * B, d), jnp.float32),
            pltpu.VMEM((2 * B, d), jnp.float32),
            pltpu.VMEM((4 * B, 128), jnp.int32),
            pltpu.VMEM((2 * B,), jnp.int32),
            pltpu.VMEM((2 * B + 16,), jnp.float32),
            pltpu.VMEM((2 * B,), jnp.float32),
            pltpu.VMEM((B + 16,), jnp.float32),
            pltpu.VMEM((B + 16,), jnp.int32),
            pltpu.VMEM((B,), jnp.int32),
            pltpu.VMEM((d,), jnp.float32),
            pltpu.VMEM((d,), jnp.float32),
            pltpu.VMEM((48,), jnp.float32),
            pltpu.VMEM((2048,), jnp.float32),
            pltpu.VMEM_SHARED((halfp,), jnp.float32),
            pltpu.VMEM_SHARED((halfp * d,), jnp.float32),
            pltpu.SemaphoreType.DMA,
            pltpu.SemaphoreType.DMA,
            pltpu.SemaphoreType.DMA,
            pltpu.SemaphoreType.DMA,
            pltpu.SemaphoreType.DMA,
            pltpu.SemaphoreType.DMA,
            pltpu.SemaphoreType.DMA,
        ],
    )
    def k(xl_hbm, xr_hbm, src_hbm, dst_hbm, ew_hbm, we_hbm, att_hbm, out_hbm,
          eidx_c, dst_c, bufL, bufR, elem_idx, src_b, w_b, w_stable,
          a_b, didx, didx_s,
          att_v, we_v, dnbuf, zbuf, denom_sp, out_sp,
          semi0, semi1, semx0, semx1, semr0, semr1, sem4):
        semi = (semi0, semi1)
        semx = (semx0, semx1)
        semr = (semr0, semr1)
        sc = lax.axis_index("c")
        tl = lax.axis_index("s")
        half_base = sc * half
        iota16 = lax.iota(jnp.int32, LANES)
        zf16 = jnp.zeros((LANES,), jnp.float32)

        for c in range(2048 // LANES):
            zbuf[pl.ds(c * LANES, LANES)] = zf16

        @pl.loop(0, rows_pt * d // 2048)
        def _(j):
            pltpu.sync_copy(
                zbuf, out_sp.at[pl.ds((tl * rows_pt) * d + j * 2048, 2048)])

        pltpu.sync_copy(zbuf.at[pl.ds(0, rows_pt)],
                        denom_sp.at[pl.ds(tl * rows_pt, rows_pt)])

        pltpu.sync_copy(att_hbm, att_v)
        pltpu.sync_copy(we_hbm, we_v)
        att_regs = [att_v[pl.ds(c * LANES, LANES)] for c in range(nd)]
        we_regs = [we_v[pl.ds(c * LANES, LANES)] for c in range(nd)]

        ebase = tl * chunk
        pltpu.sync_copy(dst_hbm.at[pl.ds(ebase, chunk)],
                        dst_c.at[pl.ds(0, chunk)])

        plsc.subcore_barrier()

        @pl.loop(0, chunk // 16, init_carry=jnp.int32(0))
        def cnt(i, cnt):
            off = i * 16
            dd = dst_c[pl.ds(off, 16)]
            m = (dd >= half_base) & (dd < half_base + half)
            plsc.store_compressed(eidx_c.at[pl.ds(cnt, 16)],
                                  ebase + off + iota16, mask=m)
            plsc.store_compressed(dst_c.at[pl.ds(cnt, 16)], dd, mask=m)
            pc = plsc.all_reduce_population_count(m)
            return cnt + jnp.max(pc)

        for j in range(2):
            eidx_c[pl.ds(cnt + j * 16, 16)] = jnp.zeros((16,), jnp.int32)
            dst_c[pl.ds(cnt + j * 16, 16)] = (
                jnp.full((16,), 1, jnp.int32) * half_base)
        nb = (cnt + B - 1) // B

        def issue_idx(b, p):
            off = b * B
            pltpu.async_copy(
                src_hbm.at[eidx_c.at[pl.ds(off, B)]],
                src_b.at[pl.ds(p * B, B)], semi[p])
            pltpu.async_copy(
                ew_hbm.at[eidx_c.at[pl.ds(off, B)]],
                w_b.at[pl.ds(p * B, B)], semi[p])

        def wait_idx(p):
            pltpu.make_async_copy(
                src_hbm.at[pl.ds(0, B)],
                src_b.at[pl.ds(p * B, B)], semi[p]).wait()
            pltpu.make_async_copy(
                ew_hbm.at[pl.ds(0, B)],
                w_b.at[pl.ds(p * B, B)], semi[p]).wait()

        def issue_rows(b, p):
            off = b * B
            pltpu.async_copy(
                xl_hbm.at[src_b.at[pl.ds(p * B, B)]],
                bufL.at[pl.ds(p * B, B)], semx[p])
            pltpu.async_copy(
                xr_hbm.at[dst_c.at[pl.ds(off, B)]],
                bufR.at[pl.ds(p * B, B)], semr[p])

        def drain_scatters(p):
            pltpu.make_async_copy(
                xl_hbm.at[pl.ds(0, B)], bufL.at[pl.ds(p * B, B)], sem4).wait()

        def pipe_block(b, p):
            off = b * B
            rb = p * B
            q = 1 - p

            @pl.when(b + 1 < nb)
            def _():
                wait_idx(q)

                @pl.when(b >= 1)
                def _():
                    drain_scatters(q)

                issue_rows(b + 1, q)

            pltpu.make_async_copy(
                xl_hbm.at[pl.ds(0, B)], bufL.at[pl.ds(rb, B)], semx[p]).wait()
            w_stable[pl.ds(0, 16)] = w_b[pl.ds(rb, 16)]

            @pl.when(b + 2 < nb)
            def _():
                issue_idx(b + 2, p)

            pltpu.make_async_copy(
                xr_hbm.at[pl.ds(0, B)], bufR.at[pl.ds(rb, B)], semr[p]).wait()

            @pl.loop(0, 16, init_carry=zf16)
            def logits(ei, logits):
                w_s = w_stable[pl.ds(ei, 16)][0]
                acc = zf16
                for c in range(nd):
                    sl = pl.ds(c * LANES, LANES)
                    v = bufL[rb + ei, sl] + bufR[rb + ei, sl] + w_s * we_regs[c]
                    v = jnp.maximum(v, 0.2 * v)
                    acc = acc + att_regs[c] * v
                lg = jnp.sum(acc)
                return jnp.where(iota16 == ei, lg, logits)

            tmask = (off + iota16) < cnt
            a_b[pl.ds(0, 16)] = jnp.where(tmask, jnp.exp(logits), 0.0)
            dl = dst_c[pl.ds(off, 16)] - half_base
            dlm = jnp.where(tmask, dl, 0)
            didx[pl.ds(0, 16)] = dlm
            didx_s[pl.ds(0, 16)] = dlm

            @pl.loop(0, B)
            def _(ei):
                a_s = a_b[pl.ds(ei, 16)][0]
                for c in range(nd):
                    sl = pl.ds(c * LANES, LANES)
                    bufL[rb + ei, sl] = a_s * bufL[rb + ei, sl]
                base = didx[pl.ds(ei, 16)][0] * d
                for c in range(nd):
                    elem_idx[2 * (rb + ei) + (c // 8),
                             pl.ds((c % 8) * 16, 16)] = (
                        base + c * LANES + iota16)

            for ei in range(B):
                pltpu.async_copy(
                    bufL.at[rb + ei, pl.ds(0, 128)],
                    out_sp.at[elem_idx.at[2 * (rb + ei)]], sem4, add=True)
                pltpu.async_copy(
                    bufL.at[rb + ei, pl.ds(128, 128)],
                    out_sp.at[elem_idx.at[2 * (rb + ei) + 1]], sem4, add=True)
            pltpu.sync_copy(a_b.at[pl.ds(0, B)],
                            denom_sp.at[didx_s], add=True)

        @pl.when(nb >= 1)
        def _():
            issue_idx(0, 0)
            wait_idx(0)
            issue_rows(0, 0)

        @pl.when(nb >= 2)
        def _():
            issue_idx(1, 1)

        @pl.loop(0, nb // 2)
        def _(j):
            pipe_block(2 * j, 0)
            pipe_block(2 * j + 1, 1)

        @pl.when(nb % 2 == 1)
        def _():
            pipe_block(nb - 1, (0))

        @pl.when(nb >= 1)
        def _():
            drain_scatters(0)

        @pl.when(nb >= 2)
        def _():
            drain_scatters(1)

        plsc.subcore_barrier()

        nchunks = half // 8
        @pl.loop(0, (nchunks + NTILES - 1) // NTILES)
        def _(j):
            idx = tl + j * NTILES

            @pl.when(idx < nchunks)
            def _():
                pltpu.sync_copy(denom_sp.at[pl.ds(idx * 8, 8)],
                                dnbuf.at[pl.ds(0, 8)])
                for r in range(8):
                    pltpu.sync_copy(
                        out_sp.at[pl.ds((idx * 8 + r) * d, d)],
                        bufR.at[r])
                dnbuf[pl.ds(16, 16)] = 1.0 / (dnbuf[pl.ds(0, 16)] + 1e-16)
                for r in range(8):
                    inv_s = dnbuf[pl.ds(16 + r, 16)][0]
                    for c in range(nd):
                        sl = pl.ds(c * LANES, LANES)
                        bufR[r, sl] = inv_s * bufR[r, sl]
                pltpu.sync_copy(
                    bufR.at[pl.ds(0, 8)],
                    out_hbm.at[pl.ds(half_base + idx * 8, 8)])

    return k(xl, xr, src, dst, ew, we, att)


def kernel(x, edge_index, edge_weight, params):
    n, d = x.shape
    src = edge_index[0]
    dst = edge_index[1]
    nl = len(params)
    h = x
    zeros_d = jnp.zeros((d,), jnp.float32)
    for i, p in enumerate(params):
        bias_prev = params[i - 1]["bias"] if i > 0 else zeros_d
        xl, xr = _dual_linear(
            h,
            bias_prev.reshape(1, d),
            p["Wl"].T,
            p["bl"].reshape(1, d),
            p["Wr"].T,
            p["br"].reshape(1, d),
            do_relu=(i > 0),
        )
        we_eff = p["We"][:, 0] if i < nl - 1 else zeros_d
        h = _gat_edge_sc(xl, xr, src, dst, edge_weight, we_eff, p["att"])
    return _bias_add(h, params[-1]["bias"].reshape(1, d))
